# Initial kernel scaffold; baseline (speedup 1.0000x reference)
import jax
import jax.numpy as jnp
from jax.experimental import pallas as pl


def kernel(inputs, envs_feat, state_t, state_s, state_e, edge_index, h_t_weights, W_ih, W_hh, b_ih, b_hh, W_dgcn, b_dgcn, W_gat, a_src, a_dst):
    raise NotImplementedError("write your pallas kernel here")


# speedup vs baseline: 72.6489x; 72.6489x over previous
import jax
import jax.numpy as jnp
from jax import lax
from jax.experimental import pallas as pl
from jax.experimental.pallas import tpu as pltpu
from jax.experimental.pallas import tpu_sc as plsc

B, N, E = 2, 10000, 320000
IN_DIM, HID, FEAT = 128, 128, 128
HEADS = 4
DH = HID // HEADS

NC, NS, L = 2, 16, 16
NP = 10240
RPT = NP // NS
EPT = E // NS


_R = 2000
_M = B * N


def _tc_pre_body(wq_ref, xin_ref, st_ref, ss_ref, se_ref, env_ref,
                 wih_ref, whh_ref, bih_ref, bhh_ref,
                 w0a_ref, w0b_ref, w1a_ref, w1b_ref, w2a_ref, w2b_ref,
                 bd_ref, wga_ref, wgb_ref, wgc_ref, a8_ref,
                 ot_ref, g0_ref, u1_ref, u2_ref, zg_ref, esed_ref):
  wv = wq_ref[0:1, 0:3]
  ex = jnp.exp(wv - jnp.max(wv))
  s = jnp.sum(ex)
  wa, wb, wc = ex[0, 0] / s, ex[0, 1] / s, ex[0, 2] / s

  x = xin_ref[...]
  h = wa * st_ref[...] + wb * ss_ref[...] + wc * se_ref[...]

  gi = jnp.dot(x, wih_ref[...], preferred_element_type=jnp.float32) + bih_ref[...]
  gh = jnp.dot(h, whh_ref[...], preferred_element_type=jnp.float32) + bhh_ref[...]
  r = jax.nn.sigmoid(gi[:, :HID] + gh[:, :HID])
  z = jax.nn.sigmoid(gi[:, HID:2 * HID] + gh[:, HID:2 * HID])
  n = jnp.tanh(gi[:, 2 * HID:] + r * gh[:, 2 * HID:])
  ot = (1.0 - z) * n + z * h
  ot_ref[...] = ot

  dot = lambda a, b: jnp.dot(a, b, preferred_element_type=jnp.float32)
  g0_ref[...] = dot(x, w0a_ref[...]) + dot(ot, w0b_ref[...]) + bd_ref[...]
  u1_ref[...] = dot(x, w1a_ref[...]) + dot(ot, w1b_ref[...])
  u2_ref[...] = dot(x, w2a_ref[...]) + dot(ot, w2b_ref[...])
  zg = dot(x, wga_ref[...]) + dot(env_ref[...], wgb_ref[...]) + dot(ot, wgc_ref[...])
  zg_ref[...] = zg
  esed_ref[...] = dot(zg, a8_ref[...])


def _tc_pre(wq, xin, st, ss, se, env, wih, whh, bih, bhh,
            w0a, w0b, w1a, w1b, w2a, w2b, bd, wga, wgb, wgc, a8):
  nblk = _M // _R
  row = pl.BlockSpec((_R, 128), lambda i: (i, 0))
  full = lambda a: pl.BlockSpec(a.shape, lambda i: tuple(0 for _ in a.shape))
  out_shapes = [
      jax.ShapeDtypeStruct((_M, HID), jnp.float32),
      jax.ShapeDtypeStruct((_M, HID), jnp.float32),
      jax.ShapeDtypeStruct((_M, HID), jnp.float32),
      jax.ShapeDtypeStruct((_M, HID), jnp.float32),
      jax.ShapeDtypeStruct((_M, HID), jnp.float32),
      jax.ShapeDtypeStruct((_M, 16), jnp.float32),
  ]
  return pl.pallas_call(
      _tc_pre_body,
      grid=(nblk,),
      in_specs=[full(wq), row, row, row, row, row,
                full(wih), full(whh), full(bih), full(bhh),
                full(w0a), full(w0b), full(w1a), full(w1b), full(w2a),
                full(w2b), full(bd), full(wga), full(wgb), full(wgc),
                full(a8)],
      out_specs=[row, row, row, row, row,
                 pl.BlockSpec((_R, 16), lambda i: (i, 0))],
      out_shape=out_shapes,
  )(wq, xin, st, ss, se, env, wih, whh, bih, bhh,
    w0a, w0b, w1a, w1b, w2a, w2b, bd, wga, wgb, wgc, a8)


_KD = 80
_WC = 80


_GDN = lax.GatherDimensionNumbers(offset_dims=(), collapsed_slice_dims=(0,),
                                  start_index_map=(0,))


def _splat(vec, j):
  idx = jnp.full((L, 1), j, jnp.int32)
  return lax.gather(vec, idx, _GDN, (1,),
                    mode=lax.GatherScatterMode.PROMISE_IN_BOUNDS)


def _sc_dgcn_body(x_hbm, extra_hbm, src_hbm, dst_hbm, z2d_hbm, z1d_hbm,
                  out_hbm,
                  acc_sh, deg_sh,
                  srcg_v, dst_v, rows_v, ones_v, stage_e, stage_a, deg_st,
                  gsem):
  c = lax.axis_index("c")
  sid = lax.axis_index("s")
  coff = c * N

  pltpu.sync_copy(z2d_hbm, acc_sh.at[pl.ds(sid * RPT, RPT)])
  pltpu.sync_copy(z1d_hbm, deg_sh.at[pl.ds(sid * RPT, RPT)])
  for i in range(_KD // L):
    ones_v[pl.ds(i * L, L)] = jnp.full((L,), 1.0, jnp.float32)
  plsc.subcore_barrier()

  ebase = sid * EPT

  def chunk(g, _):
    eb = ebase + g * _KD
    pltpu.sync_copy(src_hbm.at[pl.ds(eb, _KD)], srcg_v)
    pltpu.sync_copy(dst_hbm.at[pl.ds(eb, _KD)], dst_v)
    for i in range(_KD // L):
      sl = pl.ds(i * L, L)
      srcg_v[sl] = srcg_v[sl] + coff
    pltpu.async_copy(x_hbm.at[srcg_v], rows_v, gsem).wait()
    pltpu.sync_copy(rows_v, acc_sh.at[dst_v], add=True)
    pltpu.sync_copy(ones_v, deg_sh.at[dst_v], add=True)
    return 0

  lax.fori_loop(0, EPT // _KD, chunk, 0)
  plsc.subcore_barrier()

  w0 = sid * RPT
  wlen = jnp.minimum(RPT, N - w0)

  def wchunk(k, _):
    r0 = w0 + k * _WC
    pltpu.sync_copy(extra_hbm.at[pl.ds(coff + r0, _WC)], stage_e)
    pltpu.sync_copy(acc_sh.at[pl.ds(r0, _WC)], stage_a)
    pltpu.sync_copy(deg_sh.at[pl.ds(r0, _WC)], deg_st)
    for jg in range(_WC // L):
      dv = deg_st[pl.ds(jg * L, L)]
      inv = 1.0 / jnp.maximum(dv, 1.0)
      for j in range(L):
        iv = _splat(inv, j)
        row = jg * L + j
        for f in range(HID // L):
          fs = pl.ds(f * L, L)
          stage_a[row, fs] = stage_e[row, fs] + stage_a[row, fs] * iv
    pltpu.sync_copy(stage_a, out_hbm.at[pl.ds(coff + r0, _WC)])
    return 0

  lax.fori_loop(0, wlen // _WC, wchunk, 0)


def _sc_dgcn(x, extra, src, dst, z2d, z1d):
  mesh = plsc.VectorSubcoreMesh(core_axis_name="c", subcore_axis_name="s",
                                num_cores=NC, num_subcores=NS)
  kern = pl.kernel(
      _sc_dgcn_body,
      out_type=jax.ShapeDtypeStruct((B * N, HID), jnp.float32),
      mesh=mesh,
      scratch_types=[
          pltpu.VMEM_SHARED((NP, HID), jnp.float32),
          pltpu.VMEM_SHARED((NP,), jnp.float32),
          pltpu.VMEM((_KD,), jnp.int32),
          pltpu.VMEM((_KD,), jnp.int32),
          pltpu.VMEM((_KD, HID), jnp.float32),
          pltpu.VMEM((_KD,), jnp.float32),
          pltpu.VMEM((_WC, HID), jnp.float32),
          pltpu.VMEM((_WC, HID), jnp.float32),
          pltpu.VMEM((_WC,), jnp.float32),
          pltpu.SemaphoreType.DMA,
      ],
  )
  return kern(x, extra, src, dst, z2d, z1d)


_KG = 80


def _sc_gat_body(zg_hbm, esed_hbm, src_hbm, dst_hbm, z2d_hbm, zden_hbm,
                 num_hbm, den_hbm,
                 accz_sh, den_sh,
                 srcg_v, dst_v, dstg_v, esg_v, edg_v, zrows_v, outb_v, wb_v,
                 gsem):
  c = lax.axis_index("c")
  sid = lax.axis_index("s")
  coff = c * N

  pltpu.sync_copy(z2d_hbm, accz_sh.at[pl.ds(sid * RPT, RPT)])
  pltpu.sync_copy(zden_hbm, den_sh.at[pl.ds(sid * RPT, RPT)])
  plsc.subcore_barrier()

  ebase = sid * EPT
  lane = lax.iota(jnp.int32, L)

  def chunk(g, _):
    eb = ebase + g * _KG
    pltpu.sync_copy(src_hbm.at[pl.ds(eb, _KG)], srcg_v)
    pltpu.sync_copy(dst_hbm.at[pl.ds(eb, _KG)], dst_v)
    for i in range(_KG // L):
      sl = pl.ds(i * L, L)
      srcg_v[sl] = srcg_v[sl] + coff
      dstg_v[sl] = dst_v[sl] + coff
    pltpu.async_copy(zg_hbm.at[srcg_v], zrows_v, gsem).wait()
    pltpu.async_copy(esed_hbm.at[srcg_v], esg_v, gsem).wait()
    pltpu.async_copy(esed_hbm.at[dstg_v], edg_v, gsem).wait()
    perm = jnp.where(lane < 4, lane + 4, lane)
    for e in range(_KG):
      a = esg_v[e, :]
      b = edg_v[e, :]
      bp = lax.gather(b, perm[:, None], _GDN, (1,),
                      mode=lax.GatherScatterMode.PROMISE_IN_BOUNDS)
      sval = a + bp
      sval = jnp.maximum(sval, 0.2 * sval)
      w = jnp.exp(sval)
      wb_v[e, :] = w
      for h in range(HEADS):
        wsp = _splat(w, h)
        for j2 in range(2):
          fs = pl.ds((h * 2 + j2) * L, L)
          outb_v[e, fs] = zrows_v[e, fs] * wsp
    pltpu.sync_copy(outb_v, accz_sh.at[dst_v], add=True)
    pltpu.sync_copy(wb_v, den_sh.at[dst_v], add=True)
    return 0

  lax.fori_loop(0, EPT // _KG, chunk, 0)
  plsc.subcore_barrier()

  w0 = sid * RPT
  last = N - (NS - 1) * RPT

  @pl.when(sid < NS - 1)
  def _():
    pltpu.sync_copy(accz_sh.at[pl.ds(w0, RPT)],
                    num_hbm.at[pl.ds(coff + w0, RPT)])
    pltpu.sync_copy(den_sh.at[pl.ds(w0, RPT)],
                    den_hbm.at[pl.ds(coff + w0, RPT)])

  @pl.when(sid == NS - 1)
  def _():
    pltpu.sync_copy(accz_sh.at[pl.ds(w0, last)],
                    num_hbm.at[pl.ds(coff + w0, last)])
    pltpu.sync_copy(den_sh.at[pl.ds(w0, last)],
                    den_hbm.at[pl.ds(coff + w0, last)])


def _sc_gat(zg, esed, src, dst, z2d, zden):
  mesh = plsc.VectorSubcoreMesh(core_axis_name="c", subcore_axis_name="s",
                                num_cores=NC, num_subcores=NS)
  kern = pl.kernel(
      _sc_gat_body,
      out_type=(jax.ShapeDtypeStruct((B * N, HID), jnp.float32),
                jax.ShapeDtypeStruct((B * N, 16), jnp.float32)),
      mesh=mesh,
      scratch_types=[
          pltpu.VMEM_SHARED((NP, HID), jnp.float32),
          pltpu.VMEM_SHARED((NP, 16), jnp.float32),
          pltpu.VMEM((_KG,), jnp.int32),
          pltpu.VMEM((_KG,), jnp.int32),
          pltpu.VMEM((_KG,), jnp.int32),
          pltpu.VMEM((_KG, 16), jnp.float32),
          pltpu.VMEM((_KG, 16), jnp.float32),
          pltpu.VMEM((_KG, HID), jnp.float32),
          pltpu.VMEM((_KG, HID), jnp.float32),
          pltpu.VMEM((_KG, 16), jnp.float32),
          pltpu.SemaphoreType.DMA,
      ],
      compiler_params=pltpu.CompilerParams(use_tc_tiling_on_sc=False),
  )
  return kern(zg, esed, src, dst, z2d, zden)


_RP = 2000


def _tc_post_body(pre_ref, num_ref, den_ref, expand_ref, outs_ref, oute_ref):
  outs_ref[...] = jnp.tanh(pre_ref[...])
  den128 = jnp.dot(den_ref[...], expand_ref[...],
                   preferred_element_type=jnp.float32) + 1e-16
  x = num_ref[...] / den128
  oute_ref[...] = jnp.where(x > 0, x, jnp.exp(jnp.minimum(x, 0.0)) - 1.0)


def _tc_post(pre, num, den, expand):
  nblk = _M // _RP
  row = pl.BlockSpec((_RP, HID), lambda i: (i, 0))
  return pl.pallas_call(
      _tc_post_body,
      grid=(nblk,),
      in_specs=[row, row, pl.BlockSpec((_RP, 16), lambda i: (i, 0)),
                pl.BlockSpec(expand.shape, lambda i: (0, 0))],
      out_specs=[row, row],
      out_shape=[jax.ShapeDtypeStruct((_M, HID), jnp.float32),
                 jax.ShapeDtypeStruct((_M, HID), jnp.float32)],
  )(pre, num, den, expand)


def kernel(inputs, envs_feat, state_t, state_s, state_e, edge_index,
           h_t_weights, W_ih, W_hh, b_ih, b_hh, W_dgcn, b_dgcn, W_gat,
           a_src, a_dst):
  f32 = jnp.float32
  src = edge_index[0]
  dst = edge_index[1]

  xin = inputs.reshape(_M, IN_DIM)
  st = state_t.reshape(_M, HID)
  ss = state_s.reshape(_M, HID)
  se = state_e.reshape(_M, HID)
  env = envs_feat.reshape(_M, FEAT)

  wq = jnp.zeros((1, 8), f32).at[0, :3].set(h_t_weights)
  wihT = W_ih.T
  whhT = W_hh.T
  bih = b_ih.reshape(1, 3 * HID)
  bhh = b_hh.reshape(1, 3 * HID)
  w0a, w0b = W_dgcn[:IN_DIM], W_dgcn[IN_DIM:2 * IN_DIM]
  w1a, w1b = W_dgcn[256:256 + IN_DIM], W_dgcn[256 + IN_DIM:512]
  w2a, w2b = W_dgcn[512:512 + IN_DIM], W_dgcn[512 + IN_DIM:768]
  bd = b_dgcn.reshape(1, HID)
  wga, wgb, wgc = W_gat[:128], W_gat[128:256], W_gat[256:384]
  fidx = jnp.arange(HID) // DH
  mask = (fidx[:, None] == jnp.arange(HEADS)[None, :]).astype(f32)
  a8 = jnp.concatenate([mask * a_src.reshape(HID)[:, None],
                        mask * a_dst.reshape(HID)[:, None],
                        jnp.zeros((HID, 8), f32)], axis=1)

  z2d = jnp.zeros((RPT, HID), f32)
  z1d = jnp.zeros((RPT,), f32)
  zden = jnp.zeros((RPT, 16), f32)

  ot, g0, u1, u2, zg, esed = _tc_pre(
      wq, xin, st, ss, se, env, wihT, whhT, bih, bhh,
      w0a, w0b, w1a, w1b, w2a, w2b, bd, wga, wgb, wgc, a8)

  q = _sc_dgcn(u2, u1, src, dst, z2d, z1d)
  pre_s = _sc_dgcn(q, g0, src, dst, z2d, z1d)

  num, den = _sc_gat(zg, esed, src, dst, z2d, zden)

  expand = jnp.concatenate([mask.T, jnp.zeros((12, HID), f32)], axis=0)
  out_s, out_e = _tc_post(pre_s, num, den, expand)

  return (ot.reshape(B, N, HID),
          out_s.reshape(B, N, HID),
          out_e.reshape(B, N, HID))


# speedup vs baseline: 117.9807x; 1.6240x over previous
import jax
import jax.numpy as jnp
from jax import lax
from jax.experimental import pallas as pl
from jax.experimental.pallas import tpu as pltpu
from jax.experimental.pallas import tpu_sc as plsc

B, N, E = 2, 10000, 320000
IN_DIM, HID, FEAT = 128, 128, 128
HEADS = 4
DH = HID // HEADS

NC, NS, L = 2, 16, 16
NP = 10240
RPT = NP // NS
EPT = E // NS


_R = 2000
_M = B * N


def _tc_pre_body(wq_ref, xin_ref, st_ref, ss_ref, se_ref, env_ref,
                 wih_ref, whh_ref, bih_ref, bhh_ref,
                 w0a_ref, w0b_ref, w1a_ref, w1b_ref, w2a_ref, w2b_ref,
                 bd_ref, wga_ref, wgb_ref, wgc_ref, a8_ref,
                 ot_ref, g0_ref, u1_ref, u2_ref, zg_ref, esed_ref):
  wv = wq_ref[0:1, 0:3]
  ex = jnp.exp(wv - jnp.max(wv))
  s = jnp.sum(ex)
  wa, wb, wc = ex[0, 0] / s, ex[0, 1] / s, ex[0, 2] / s

  x = xin_ref[...]
  h = wa * st_ref[...] + wb * ss_ref[...] + wc * se_ref[...]

  gi = jnp.dot(x, wih_ref[...], preferred_element_type=jnp.float32) + bih_ref[...]
  gh = jnp.dot(h, whh_ref[...], preferred_element_type=jnp.float32) + bhh_ref[...]
  r = jax.nn.sigmoid(gi[:, :HID] + gh[:, :HID])
  z = jax.nn.sigmoid(gi[:, HID:2 * HID] + gh[:, HID:2 * HID])
  n = jnp.tanh(gi[:, 2 * HID:] + r * gh[:, 2 * HID:])
  ot = (1.0 - z) * n + z * h
  ot_ref[...] = ot

  dot = lambda a, b: jnp.dot(a, b, preferred_element_type=jnp.float32)
  g0_ref[...] = dot(x, w0a_ref[...]) + dot(ot, w0b_ref[...]) + bd_ref[...]
  u1_ref[...] = dot(x, w1a_ref[...]) + dot(ot, w1b_ref[...])
  u2_ref[...] = dot(x, w2a_ref[...]) + dot(ot, w2b_ref[...])
  zg = dot(x, wga_ref[...]) + dot(env_ref[...], wgb_ref[...]) + dot(ot, wgc_ref[...])
  zg_ref[...] = zg
  esed_ref[...] = dot(zg, a8_ref[...])


def _tc_pre(wq, xin, st, ss, se, env, wih, whh, bih, bhh,
            w0a, w0b, w1a, w1b, w2a, w2b, bd, wga, wgb, wgc, a8):
  nblk = _M // _R
  row = pl.BlockSpec((_R, 128), lambda i: (i, 0))
  full = lambda a: pl.BlockSpec(a.shape, lambda i: tuple(0 for _ in a.shape))
  out_shapes = [
      jax.ShapeDtypeStruct((_M, HID), jnp.float32),
      jax.ShapeDtypeStruct((_M, HID), jnp.float32),
      jax.ShapeDtypeStruct((_M, HID), jnp.float32),
      jax.ShapeDtypeStruct((_M, HID), jnp.float32),
      jax.ShapeDtypeStruct((_M, HID), jnp.float32),
      jax.ShapeDtypeStruct((_M, 16), jnp.float32),
  ]
  return pl.pallas_call(
      _tc_pre_body,
      grid=(nblk,),
      in_specs=[full(wq), row, row, row, row, row,
                full(wih), full(whh), full(bih), full(bhh),
                full(w0a), full(w0b), full(w1a), full(w1b), full(w2a),
                full(w2b), full(bd), full(wga), full(wgb), full(wgc),
                full(a8)],
      out_specs=[row, row, row, row, row,
                 pl.BlockSpec((_R, 16), lambda i: (i, 0))],
      out_shape=out_shapes,
  )(wq, xin, st, ss, se, env, wih, whh, bih, bhh,
    w0a, w0b, w1a, w1b, w2a, w2b, bd, wga, wgb, wgc, a8)


_KD = 80
_WC = 80


_GDN = lax.GatherDimensionNumbers(offset_dims=(), collapsed_slice_dims=(0,),
                                  start_index_map=(0,))


def _splat(vec, j):
  idx = jnp.full((L, 1), j, jnp.int32)
  return lax.gather(vec, idx, _GDN, (1,),
                    mode=lax.GatherScatterMode.PROMISE_IN_BOUNDS)


def _sc_dgcn_body(x_hbm, extra_hbm, src_hbm, dst_hbm, z2d_hbm, z1d_hbm,
                  out_hbm,
                  acc_sh, deg_sh,
                  idx_v, srcg_v, dstl_v, rows_v, ones_v, deg_st,
                  isem, g0sem, g1sem, ssem, dsem):
  c = lax.axis_index("c")
  sid = lax.axis_index("s")
  coff = c * N

  pltpu.sync_copy(z2d_hbm, acc_sh.at[pl.ds(sid * RPT, RPT)])
  pltpu.sync_copy(z1d_hbm, deg_sh.at[pl.ds(sid * RPT, RPT)])
  for i in range(_KD // L):
    ones_v[pl.ds(i * L, L)] = jnp.full((L,), 1.0, jnp.float32)
  plsc.subcore_barrier()

  ebase = sid * EPT
  gsems = (g0sem, g1sem)

  def pair(j, _):
    eb0 = ebase + (2 * j) * _KD
    ids = []
    for b in range(2):
      eb = eb0 + b * _KD
      ids.append(pltpu.async_copy(src_hbm.at[pl.ds(eb, _KD)],
                                  idx_v.at[b, 0], isem))
      ids.append(pltpu.async_copy(dst_hbm.at[pl.ds(eb, _KD)],
                                  idx_v.at[b, 1], isem))
    for d in ids:
      d.wait()
    gds = []
    for b in range(2):
      for i in range(_KD // L):
        sl = pl.ds(i * L, L)
        srcg_v[b, sl] = idx_v[b, 0, sl] + coff
        dstl_v[b, sl] = idx_v[b, 1, sl]
      gds.append(pltpu.async_copy(x_hbm.at[srcg_v.at[b]], rows_v.at[b],
                                  gsems[b]))
    sds = []
    for b in range(2):
      gds[b].wait()
      sds.append(pltpu.async_copy(rows_v.at[b], acc_sh.at[dstl_v.at[b]],
                                  ssem, add=True))
      sds.append(pltpu.async_copy(ones_v, deg_sh.at[dstl_v.at[b]],
                                  dsem, add=True))
    for d in sds:
      d.wait()
    return 0

  lax.fori_loop(0, EPT // (2 * _KD), pair, 0)
  plsc.subcore_barrier()

  w0 = sid * RPT
  wlen = jnp.minimum(RPT, N - w0)
  stage_e = rows_v.at[0]
  stage_a = rows_v.at[1]

  def wchunk(k, _):
    r0 = w0 + k * _WC
    pltpu.sync_copy(extra_hbm.at[pl.ds(coff + r0, _WC)], stage_e)
    pltpu.sync_copy(acc_sh.at[pl.ds(r0, _WC)], stage_a)
    pltpu.sync_copy(deg_sh.at[pl.ds(r0, _WC)], deg_st)
    for jg in range(_WC // L):
      dv = deg_st[pl.ds(jg * L, L)]
      inv = 1.0 / jnp.maximum(dv, 1.0)
      for j in range(L):
        iv = _splat(inv, j)
        row = jg * L + j
        for f in range(HID // L):
          fs = pl.ds(f * L, L)
          stage_a[row, fs] = stage_e[row, fs] + stage_a[row, fs] * iv
    pltpu.sync_copy(stage_a, out_hbm.at[pl.ds(coff + r0, _WC)])
    return 0

  lax.fori_loop(0, wlen // _WC, wchunk, 0)


def _sc_dgcn(x, extra, src, dst, z2d, z1d):
  mesh = plsc.VectorSubcoreMesh(core_axis_name="c", subcore_axis_name="s",
                                num_cores=NC, num_subcores=NS)
  kern = pl.kernel(
      _sc_dgcn_body,
      out_type=jax.ShapeDtypeStruct((B * N, HID), jnp.float32),
      mesh=mesh,
      scratch_types=[
          pltpu.VMEM_SHARED((NP, HID), jnp.float32),
          pltpu.VMEM_SHARED((NP,), jnp.float32),
          pltpu.VMEM((2, 2, _KD), jnp.int32),
          pltpu.VMEM((2, _KD), jnp.int32),
          pltpu.VMEM((2, _KD), jnp.int32),
          pltpu.VMEM((2, _KD, HID), jnp.float32),
          pltpu.VMEM((_KD,), jnp.float32),
          pltpu.VMEM((_WC,), jnp.float32),
          pltpu.SemaphoreType.DMA,
          pltpu.SemaphoreType.DMA,
          pltpu.SemaphoreType.DMA,
          pltpu.SemaphoreType.DMA,
          pltpu.SemaphoreType.DMA,
      ],
  )
  return kern(x, extra, src, dst, z2d, z1d)


_KG = 80


def _sc_gat_body(zg_hbm, esed_hbm, src_hbm, dst_hbm, z2d_hbm, zden_hbm,
                 num_hbm, den_hbm,
                 accz_sh, den_sh,
                 idx_v, srcg_v, dstg_v, dstl_v, esg_v, edg_v, zrows_v, wb_v,
                 isem, gz0, gz1, ge0, ge1, gd0, gd1, ssem, wsem):
  c = lax.axis_index("c")
  sid = lax.axis_index("s")
  coff = c * N

  pltpu.sync_copy(z2d_hbm, accz_sh.at[pl.ds(sid * RPT, RPT)])
  pltpu.sync_copy(zden_hbm, den_sh.at[pl.ds(sid * RPT, RPT)])
  plsc.subcore_barrier()

  ebase = sid * EPT
  lane = lax.iota(jnp.int32, L)
  gzs, ges, gds_ = (gz0, gz1), (ge0, ge1), (gd0, gd1)

  def pair(j, _):
    eb0 = ebase + (2 * j) * _KG
    ids = []
    for b in range(2):
      eb = eb0 + b * _KG
      ids.append(pltpu.async_copy(src_hbm.at[pl.ds(eb, _KG)],
                                  idx_v.at[b, 0], isem))
      ids.append(pltpu.async_copy(dst_hbm.at[pl.ds(eb, _KG)],
                                  idx_v.at[b, 1], isem))
    for d in ids:
      d.wait()
    descs = []
    for b in range(2):
      for i in range(_KG // L):
        sl = pl.ds(i * L, L)
        srcg_v[b, sl] = idx_v[b, 0, sl] + coff
        dstg_v[b, sl] = idx_v[b, 1, sl] + coff
        dstl_v[b, sl] = idx_v[b, 1, sl]
      descs.append((
          pltpu.async_copy(zg_hbm.at[srcg_v.at[b]], zrows_v.at[b], gzs[b]),
          pltpu.async_copy(esed_hbm.at[srcg_v.at[b]], esg_v.at[b], ges[b]),
          pltpu.async_copy(esed_hbm.at[dstg_v.at[b]], edg_v.at[b], gds_[b]),
      ))
    perm = jnp.where(lane < 4, lane + 4, lane)
    sds = []
    for b in range(2):
      for d in descs[b]:
        d.wait()
      for e in range(_KG):
        a = esg_v[b, e, :]
        bb = edg_v[b, e, :]
        bp = lax.gather(bb, perm[:, None], _GDN, (1,),
                        mode=lax.GatherScatterMode.PROMISE_IN_BOUNDS)
        sval = a + bp
        sval = jnp.maximum(sval, 0.2 * sval)
        w = jnp.exp(sval)
        wb_v[b, e, :] = w
        for h in range(HEADS):
          wsp = _splat(w, h)
          for j2 in range(2):
            fs = pl.ds((h * 2 + j2) * L, L)
            zrows_v[b, e, fs] = zrows_v[b, e, fs] * wsp
      sds.append(pltpu.async_copy(zrows_v.at[b], accz_sh.at[dstl_v.at[b]],
                                  ssem, add=True))
      sds.append(pltpu.async_copy(wb_v.at[b], den_sh.at[dstl_v.at[b]],
                                  wsem, add=True))
    for d in sds:
      d.wait()
    return 0

  lax.fori_loop(0, EPT // (2 * _KG), pair, 0)
  plsc.subcore_barrier()

  w0 = sid * RPT
  last = N - (NS - 1) * RPT

  @pl.when(sid < NS - 1)
  def _():
    pltpu.sync_copy(accz_sh.at[pl.ds(w0, RPT)],
                    num_hbm.at[pl.ds(coff + w0, RPT)])
    pltpu.sync_copy(den_sh.at[pl.ds(w0, RPT)],
                    den_hbm.at[pl.ds(coff + w0, RPT)])

  @pl.when(sid == NS - 1)
  def _():
    pltpu.sync_copy(accz_sh.at[pl.ds(w0, last)],
                    num_hbm.at[pl.ds(coff + w0, last)])
    pltpu.sync_copy(den_sh.at[pl.ds(w0, last)],
                    den_hbm.at[pl.ds(coff + w0, last)])


def _sc_gat(zg, esed, src, dst, z2d, zden):
  mesh = plsc.VectorSubcoreMesh(core_axis_name="c", subcore_axis_name="s",
                                num_cores=NC, num_subcores=NS)
  kern = pl.kernel(
      _sc_gat_body,
      out_type=(jax.ShapeDtypeStruct((B * N, HID), jnp.float32),
                jax.ShapeDtypeStruct((B * N, 16), jnp.float32)),
      mesh=mesh,
      scratch_types=[
          pltpu.VMEM_SHARED((NP, HID), jnp.float32),
          pltpu.VMEM_SHARED((NP, 16), jnp.float32),
          pltpu.VMEM((2, 2, _KG), jnp.int32),
          pltpu.VMEM((2, _KG), jnp.int32),
          pltpu.VMEM((2, _KG), jnp.int32),
          pltpu.VMEM((2, _KG), jnp.int32),
          pltpu.VMEM((2, _KG, 16), jnp.float32),
          pltpu.VMEM((2, _KG, 16), jnp.float32),
          pltpu.VMEM((2, _KG, HID), jnp.float32),
          pltpu.VMEM((2, _KG, 16), jnp.float32),
      ] + [pltpu.SemaphoreType.DMA] * 9,
      compiler_params=pltpu.CompilerParams(use_tc_tiling_on_sc=False),
  )
  return kern(zg, esed, src, dst, z2d, zden)


_RP = 2000


def _tc_post_body(pre_ref, num_ref, den_ref, expand_ref, outs_ref, oute_ref):
  outs_ref[...] = jnp.tanh(pre_ref[...])
  den128 = jnp.dot(den_ref[...], expand_ref[...],
                   preferred_element_type=jnp.float32) + 1e-16
  x = num_ref[...] / den128
  oute_ref[...] = jnp.where(x > 0, x, jnp.exp(jnp.minimum(x, 0.0)) - 1.0)


def _tc_post(pre, num, den, expand):
  nblk = _M // _RP
  row = pl.BlockSpec((_RP, HID), lambda i: (i, 0))
  return pl.pallas_call(
      _tc_post_body,
      grid=(nblk,),
      in_specs=[row, row, pl.BlockSpec((_RP, 16), lambda i: (i, 0)),
                pl.BlockSpec(expand.shape, lambda i: (0, 0))],
      out_specs=[row, row],
      out_shape=[jax.ShapeDtypeStruct((_M, HID), jnp.float32),
                 jax.ShapeDtypeStruct((_M, HID), jnp.float32)],
  )(pre, num, den, expand)


def kernel(inputs, envs_feat, state_t, state_s, state_e, edge_index,
           h_t_weights, W_ih, W_hh, b_ih, b_hh, W_dgcn, b_dgcn, W_gat,
           a_src, a_dst):
  f32 = jnp.float32
  src = edge_index[0]
  dst = edge_index[1]

  xin = inputs.reshape(_M, IN_DIM)
  st = state_t.reshape(_M, HID)
  ss = state_s.reshape(_M, HID)
  se = state_e.reshape(_M, HID)
  env = envs_feat.reshape(_M, FEAT)

  wq = jnp.zeros((1, 8), f32).at[0, :3].set(h_t_weights)
  wihT = W_ih.T
  whhT = W_hh.T
  bih = b_ih.reshape(1, 3 * HID)
  bhh = b_hh.reshape(1, 3 * HID)
  w0a, w0b = W_dgcn[:IN_DIM], W_dgcn[IN_DIM:2 * IN_DIM]
  w1a, w1b = W_dgcn[256:256 + IN_DIM], W_dgcn[256 + IN_DIM:512]
  w2a, w2b = W_dgcn[512:512 + IN_DIM], W_dgcn[512 + IN_DIM:768]
  bd = b_dgcn.reshape(1, HID)
  wga, wgb, wgc = W_gat[:128], W_gat[128:256], W_gat[256:384]
  fidx = jnp.arange(HID) // DH
  mask = (fidx[:, None] == jnp.arange(HEADS)[None, :]).astype(f32)
  a8 = jnp.concatenate([mask * a_src.reshape(HID)[:, None],
                        mask * a_dst.reshape(HID)[:, None],
                        jnp.zeros((HID, 8), f32)], axis=1)

  z2d = jnp.zeros((RPT, HID), f32)
  z1d = jnp.zeros((RPT,), f32)
  zden = jnp.zeros((RPT, 16), f32)

  ot, g0, u1, u2, zg, esed = _tc_pre(
      wq, xin, st, ss, se, env, wihT, whhT, bih, bhh,
      w0a, w0b, w1a, w1b, w2a, w2b, bd, wga, wgb, wgc, a8)

  q = _sc_dgcn(u2, u1, src, dst, z2d, z1d)
  pre_s = _sc_dgcn(q, g0, src, dst, z2d, z1d)

  num, den = _sc_gat(zg, esed, src, dst, z2d, zden)

  expand = jnp.concatenate([mask.T, jnp.zeros((12, HID), f32)], axis=0)
  out_s, out_e = _tc_post(pre_s, num, den, expand)

  return (ot.reshape(B, N, HID),
          out_s.reshape(B, N, HID),
          out_e.reshape(B, N, HID))
